# baseline (speedup 1.0000x reference)
import functools

import jax
import jax.numpy as jnp
from jax import lax
from jax.experimental import pallas as pl
from jax.experimental.pallas import tpu as pltpu
from jax.experimental.pallas import tpu_sc as plsc

D = 64
IDX_ROW = 128
CHUNK = 128
NBUF = 4


@functools.cache
def _make_gather(B):
    info = plsc.get_sparse_core_info()
    nw = info.num_cores * info.num_subcores
    H = B // 2
    assert H % (nw * CHUNK * NBUF) == 0
    h_per_w = H // nw
    n_chunks = h_per_w // CHUNK
    n_idx_rows = h_per_w // IDX_ROW

    mesh = plsc.VectorSubcoreMesh(core_axis_name="c", subcore_axis_name="s")

    @functools.partial(
        pl.kernel,
        mesh=mesh,
        out_type=jax.ShapeDtypeStruct((H, 2 * D), jnp.float32),
        scratch_types=[
            pltpu.VMEM((n_idx_rows, IDX_ROW), jnp.int32),
            pltpu.VMEM((n_idx_rows, IDX_ROW), jnp.int32),
            pltpu.VMEM((NBUF, 2, CHUNK, D), jnp.float32),
        ]
        + [pltpu.SemaphoreType.DMA] * (2 * NBUF),
        compiler_params=pltpu.CompilerParams(use_tc_tiling_on_sc=False),
    )
    def emb(xe_hbm, xo_hbm, table_hbm, out_hbm, idx_e, idx_o, rows, *sems):
        sem_g, sem_w = sems[:NBUF], sems[NBUF:]
        wid = lax.axis_index("s") * info.num_cores + lax.axis_index("c")
        row0 = wid * n_idx_rows
        out0 = wid * h_per_w

        pltpu.sync_copy(xe_hbm.at[pl.ds(row0, n_idx_rows)], idx_e)
        pltpu.sync_copy(xo_hbm.at[pl.ds(row0, n_idx_rows)], idx_o)

        def fire_gather(c, b):
            pltpu.async_copy(table_hbm.at[idx_e.at[c]], rows.at[b].at[0], sem_g[b])
            pltpu.async_copy(table_hbm.at[idx_o.at[c]], rows.at[b].at[1], sem_g[b])

        def wait_gather(b):
            pltpu.make_async_copy(
                table_hbm.at[pl.ds(0, CHUNK)], rows.at[b].at[0], sem_g[b]
            ).wait()
            pltpu.make_async_copy(
                table_hbm.at[pl.ds(0, CHUNK)], rows.at[b].at[1], sem_g[b]
            ).wait()

        def _write_copies(c, b):
            dst = out_hbm.at[pl.ds(out0 + c * CHUNK, CHUNK)]
            return (
                pltpu.make_async_copy(
                    rows.at[b].at[0], dst.at[:, pl.ds(0, D)], sem_w[b]
                ),
                pltpu.make_async_copy(
                    rows.at[b].at[1], dst.at[:, pl.ds(D, D)], sem_w[b]
                ),
            )

        def fire_write(c, b):
            for cp in _write_copies(c, b):
                cp.start()

        def wait_write(c, b):
            for cp in _write_copies(c, b):
                cp.wait()

        fire_gather(0, 0)
        fire_gather(1, 1)

        def group(t, carry):
            for b in range(NBUF):
                c = t * NBUF + b
                wait_gather(b)
                fire_write(c, b)
                s2 = (b + 2) % NBUF

                @pl.when(c + 2 < n_chunks)
                def _():
                    @pl.when(c >= 2)
                    def _():
                        wait_write(c - 2, s2)

                    fire_gather(c + 2, s2)

            return carry

        lax.fori_loop(0, n_chunks // NBUF, group, 0)

        for b in range(NBUF):
            wait_write(n_chunks - NBUF + b, b)

    return emb


def kernel(x, table):
    orig_shape = x.shape
    B = x.size
    xp = x.reshape(B // 2, 2).astype(jnp.int32)
    xe = xp[:, 0].reshape(-1, IDX_ROW)
    xo = xp[:, 1].reshape(-1, IDX_ROW)
    out = _make_gather(B)(xe, xo, table)
    return out.reshape(*orig_shape, D)


# speedup vs baseline: 1.3983x; 1.3983x over previous
import functools

import jax
import jax.numpy as jnp
from jax import lax
from jax.experimental import pallas as pl
from jax.experimental.pallas import tpu as pltpu
from jax.experimental.pallas import tpu_sc as plsc

D = 64
IDX_ROW = 128
K = 2
CHUNK = K * IDX_ROW
NBUF = 4


@functools.cache
def _make_gather(B):
    info = plsc.get_sparse_core_info()
    nw = info.num_cores * info.num_subcores
    assert B % (nw * CHUNK * NBUF) == 0
    b_per_w = B // nw
    n_chunks = b_per_w // CHUNK
    n_groups = n_chunks // NBUF
    n_idx_rows = b_per_w // IDX_ROW

    mesh = plsc.VectorSubcoreMesh(core_axis_name="c", subcore_axis_name="s")

    @functools.partial(
        pl.kernel,
        mesh=mesh,
        out_type=jax.ShapeDtypeStruct((B // IDX_ROW, IDX_ROW, D), jnp.float32),
        scratch_types=[
            pltpu.VMEM((n_idx_rows, IDX_ROW), jnp.int32),
            pltpu.VMEM((NBUF, K, IDX_ROW, D), jnp.float32),
        ]
        + [pltpu.SemaphoreType.DMA] * (2 * NBUF),
        compiler_params=pltpu.CompilerParams(use_tc_tiling_on_sc=False),
    )
    def emb(x_hbm, table_hbm, out_hbm, idx_all, rows, *sems):
        sem_g, sem_w = sems[:NBUF], sems[NBUF:]
        wid = lax.axis_index("s") * info.num_cores + lax.axis_index("c")
        row0 = wid * n_idx_rows

        pltpu.sync_copy(x_hbm.at[pl.ds(row0, n_idx_rows)], idx_all)

        def fire_gather(c, b):
            for j in range(K):
                pltpu.async_copy(
                    table_hbm.at[idx_all.at[c * K + j]],
                    rows.at[b].at[j],
                    sem_g[b],
                )

        def wait_gather(b):
            for j in range(K):
                pltpu.make_async_copy(
                    table_hbm.at[pl.ds(0, IDX_ROW)], rows.at[b].at[j], sem_g[b]
                ).wait()

        def fire_write(c, b):
            pltpu.async_copy(
                rows.at[b], out_hbm.at[pl.ds(row0 + c * K, K)], sem_w[b]
            )

        def wait_write(c, b):
            pltpu.make_async_copy(
                rows.at[b], out_hbm.at[pl.ds(row0 + c * K, K)], sem_w[b]
            ).wait()

        fire_gather(0, 0)
        fire_gather(1, 1)

        def group(t, carry):
            for b in range(NBUF):
                c = t * NBUF + b
                wait_gather(b)
                fire_write(c, b)
                s2 = (b + 2) % NBUF

                @pl.when(c + 2 < n_chunks)
                def _():
                    @pl.when(c >= 2)
                    def _():
                        wait_write(c - 2, s2)

                    fire_gather(c + 2, s2)

            return carry

        lax.fori_loop(0, n_groups, group, 0)

        for b in range(NBUF):
            wait_write(n_chunks - NBUF + b, b)

    return emb


def kernel(x, table):
    orig_shape = x.shape
    B = x.size
    x2d = x.reshape(B // IDX_ROW, IDX_ROW).astype(jnp.int32)
    out = _make_gather(B)(x2d, table)
    return out.reshape(*orig_shape, D)
